# baseline (speedup 1.0000x reference)
import jax
import jax.numpy as jnp
from jax import lax
from jax.experimental import pallas as pl
from jax.experimental.pallas import tpu as pltpu

_T = 2048
_D = 768
_I = 3072
_E = 8
_IB = 1024
_NI = _I // _IB


def _gelu(v):
    return 0.5 * v * (1.0 + lax.erf(v * (2.0 ** -0.5)))


def _moe_body(x_ref, wr1_ref, wr2_ref, wi_ref, bi_ref, wo_ref, bo_ref,
              out_ref, c_scr, w0_scr, w1_scr, any_scr, const_scr):
    e = pl.program_id(0)
    i = pl.program_id(1)

    @pl.when(jnp.logical_and(e == 0, i == 0))
    def _router():
        x = x_ref[...]
        h = _gelu(lax.dot_general(x, wr1_ref[...], (((1,), (1,)), ((), ())),
                                  preferred_element_type=jnp.float32))
        logits = lax.dot_general(h, wr2_ref[...], (((1,), (1,)), ((), ())),
                                 preferred_element_type=jnp.float32)
        m = jnp.max(logits, axis=-1, keepdims=True)
        ex = jnp.exp(logits - m)
        p = ex / jnp.sum(ex, axis=-1, keepdims=True)

        iota = lax.broadcasted_iota(jnp.int32, (_T, _E), 1)
        big = jnp.int32(_E + 1)
        m0 = jnp.max(p, axis=-1, keepdims=True)
        i0 = jnp.min(jnp.where(p == m0, iota, big), axis=-1, keepdims=True)
        oh0 = (iota == i0).astype(jnp.float32)
        p2 = jnp.where(iota == i0, -1.0, p)
        m1 = jnp.max(p2, axis=-1, keepdims=True)
        i1 = jnp.min(jnp.where(p2 == m1, iota, big), axis=-1, keepdims=True)
        oh1 = (iota == i1).astype(jnp.float32)

        c_scr[...] = m0 * oh0 + m1 * oh1
        w0_scr[...] = m0
        w1_scr[...] = m1
        any_scr[0:1, :] = jnp.max(oh0, axis=0, keepdims=True)
        any_scr[1:2, :] = jnp.max(oh1, axis=0, keepdims=True)
        const_scr[...] = jnp.zeros((_E, _D), jnp.float32)
        out_ref[...] = jnp.zeros((_T, _D), jnp.float32)

    x = x_ref[...]

    iota = lax.broadcasted_iota(jnp.int32, (_T, _E), 1)
    ce = jnp.sum(jnp.where(iota == e, c_scr[...], 0.0), axis=-1, keepdims=True)
    _H = _IB // 2
    acc = None
    for h in range(2):
        wi_h = wi_ref[0, h * _H:(h + 1) * _H, :]
        wo_h = wo_ref[0, :, h * _H:(h + 1) * _H]
        bi_h = bi_ref[0, 0, :, h * _H:(h + 1) * _H]
        pre_h = lax.dot_general(x, wi_h, (((1,), (1,)), ((), ())),
                                preferred_element_type=jnp.float32) + bi_h
        act_h = _gelu(pre_h) * ce
        y_h = lax.dot_general(act_h, wo_h, (((1,), (1,)), ((), ())),
                              preferred_element_type=jnp.float32)
        acc = y_h if acc is None else acc + y_h
    out_ref[...] += acc

    g = _gelu(bi_ref[0, 0])
    rowc = lax.dot_general(g, wo_ref[0], (((1,), (1,)), ((), ())),
                           preferred_element_type=jnp.float32)
    const_scr[pl.ds(e, 1), :] += rowc

    @pl.when(jnp.logical_and(e == _E - 1, i == _NI - 1))
    def _correction():
        constmm = const_scr[...]
        const_full = constmm + bo_ref[...]
        r0 = lax.dot_general(any_scr[0:1, :], const_full, (((1,), (0,)), ((), ())),
                             preferred_element_type=jnp.float32)
        r1 = lax.dot_general(any_scr[1:2, :], const_full, (((1,), (0,)), ((), ())),
                             preferred_element_type=jnp.float32)
        corr = lax.dot_general(c_scr[...], constmm, (((1,), (0,)), ((), ())),
                               preferred_element_type=jnp.float32)
        out_ref[...] += w0_scr[...] * r0 + w1_scr[...] * r1 - corr


def kernel(x, Wr1, Wr2, Wi, bi, Wo, bo):
    B, T, D = x.shape
    xf = x.reshape(T, D)
    out = pl.pallas_call(
        _moe_body,
        grid=(_E, _NI),
        in_specs=[
            pl.BlockSpec((_T, _D), lambda e, i: (0, 0)),
            pl.BlockSpec((_D // 2, _D), lambda e, i: (0, 0)),
            pl.BlockSpec((_E, _D // 2), lambda e, i: (0, 0)),
            pl.BlockSpec((1, _IB, _D), lambda e, i: (e, i, 0)),
            pl.BlockSpec((1, 1, 1, _IB), lambda e, i: (e, i, 0, 0)),
            pl.BlockSpec((1, _D, _IB), lambda e, i: (e, 0, i)),
            pl.BlockSpec((_E, _D), lambda e, i: (0, 0)),
        ],
        out_specs=pl.BlockSpec((_T, _D), lambda e, i: (0, 0)),
        out_shape=jax.ShapeDtypeStruct((T, D), jnp.float32),
        scratch_shapes=[
            pltpu.VMEM((_T, _E), jnp.float32),
            pltpu.VMEM((_T, 1), jnp.float32),
            pltpu.VMEM((_T, 1), jnp.float32),
            pltpu.VMEM((2, _E), jnp.float32),
            pltpu.VMEM((_E, _D), jnp.float32),
        ],
    )(xf, Wr1, Wr2, Wi, bi.reshape(_E, _NI, 1, _IB), Wo, bo)
    return out.reshape(B, T, D)


# speedup vs baseline: 1.0984x; 1.0984x over previous
import jax
import jax.numpy as jnp
from jax import lax
from jax.experimental import pallas as pl
from jax.experimental.pallas import tpu as pltpu

_T = 2048
_D = 768
_I = 3072
_E = 8
_IB = 1024
_NI = _I // _IB


def _gelu(v):
    return 0.5 * v * (1.0 + lax.erf(v * (2.0 ** -0.5)))


def _moe_body(x_ref, wr1_ref, wr2_ref, wi_ref, bi_ref, wo_ref, bo_ref,
              out_ref, c_scr, w0_scr, w1_scr, any_scr, const_scr):
    e = pl.program_id(0)
    i = pl.program_id(1)

    @pl.when(jnp.logical_and(e == 0, i == 0))
    def _router():
        x = x_ref[...]
        h = _gelu(lax.dot_general(x, wr1_ref[...], (((1,), (1,)), ((), ())),
                                  preferred_element_type=jnp.float32))
        logits = lax.dot_general(h, wr2_ref[...], (((1,), (1,)), ((), ())),
                                 preferred_element_type=jnp.float32)
        m = jnp.max(logits, axis=-1, keepdims=True)
        ex = jnp.exp(logits - m)
        p = ex / jnp.sum(ex, axis=-1, keepdims=True)

        iota = lax.broadcasted_iota(jnp.int32, (_T, _E), 1)
        big = jnp.int32(_E + 1)
        m0 = jnp.max(p, axis=-1, keepdims=True)
        i0 = jnp.min(jnp.where(p == m0, iota, big), axis=-1, keepdims=True)
        oh0 = (iota == i0).astype(jnp.float32)
        p2 = jnp.where(iota == i0, -1.0, p)
        m1 = jnp.max(p2, axis=-1, keepdims=True)
        i1 = jnp.min(jnp.where(p2 == m1, iota, big), axis=-1, keepdims=True)
        oh1 = (iota == i1).astype(jnp.float32)

        c_scr[...] = m0 * oh0 + m1 * oh1
        w0_scr[...] = m0
        w1_scr[...] = m1
        any_scr[0:1, :] = jnp.max(oh0, axis=0, keepdims=True)
        any_scr[1:2, :] = jnp.max(oh1, axis=0, keepdims=True)
        const_scr[...] = jnp.zeros((_E, _D), jnp.float32)
        out_ref[...] = jnp.zeros((_T, _D), jnp.float32)

    x = x_ref[...]
    wi = wi_ref[0]
    wo = wo_ref[0]
    bi_row = bi_ref[0, 0]

    pre = lax.dot_general(x, wi, (((1,), (1,)), ((), ())),
                          preferred_element_type=jnp.float32) + bi_row
    act = _gelu(pre)
    iota = lax.broadcasted_iota(jnp.int32, (_T, _E), 1)
    ce = jnp.sum(jnp.where(iota == e, c_scr[...], 0.0), axis=-1, keepdims=True)
    out_ref[...] += lax.dot_general(act * ce, wo, (((1,), (1,)), ((), ())),
                                    preferred_element_type=jnp.float32)

    g = _gelu(bi_row)
    rowc = lax.dot_general(g, wo, (((1,), (1,)), ((), ())),
                           preferred_element_type=jnp.float32)
    const_scr[pl.ds(e, 1), :] += rowc

    @pl.when(jnp.logical_and(e == _E - 1, i == _NI - 1))
    def _correction():
        constmm = const_scr[...]
        const_full = constmm + bo_ref[...]
        r0 = lax.dot_general(any_scr[0:1, :], const_full, (((1,), (0,)), ((), ())),
                             preferred_element_type=jnp.float32)
        r1 = lax.dot_general(any_scr[1:2, :], const_full, (((1,), (0,)), ((), ())),
                             preferred_element_type=jnp.float32)
        corr = lax.dot_general(c_scr[...], constmm, (((1,), (0,)), ((), ())),
                               preferred_element_type=jnp.float32)
        out_ref[...] += w0_scr[...] * r0 + w1_scr[...] * r1 - corr


def kernel(x, Wr1, Wr2, Wi, bi, Wo, bo):
    B, T, D = x.shape
    xf = x.reshape(T, D)
    out = pl.pallas_call(
        _moe_body,
        grid=(_E, _NI),
        in_specs=[
            pl.BlockSpec((_T, _D), lambda e, i: (0, 0)),
            pl.BlockSpec((_D // 2, _D), lambda e, i: (0, 0)),
            pl.BlockSpec((_E, _D // 2), lambda e, i: (0, 0)),
            pl.BlockSpec((1, _IB, _D), lambda e, i: (e, i, 0)),
            pl.BlockSpec((1, 1, 1, _IB), lambda e, i: (e, i, 0, 0)),
            pl.BlockSpec((1, _D, _IB), lambda e, i: (e, 0, i)),
            pl.BlockSpec((_E, _D), lambda e, i: (0, 0)),
        ],
        out_specs=pl.BlockSpec((_T, _D), lambda e, i: (0, 0)),
        out_shape=jax.ShapeDtypeStruct((T, D), jnp.float32),
        scratch_shapes=[
            pltpu.VMEM((_T, _E), jnp.float32),
            pltpu.VMEM((_T, 1), jnp.float32),
            pltpu.VMEM((_T, 1), jnp.float32),
            pltpu.VMEM((2, _E), jnp.float32),
            pltpu.VMEM((_E, _D), jnp.float32),
        ],
    )(xf, Wr1, Wr2, Wi, bi.reshape(_E, _NI, 1, _IB), Wo, bo)
    return out.reshape(B, T, D)
